# baseline (speedup 1.0000x reference)
import functools

import jax
import jax.numpy as jnp
from jax import lax
from jax.experimental import pallas as pl
from jax.experimental.pallas import tpu as pltpu
from jax.experimental.pallas import tpu_sc as plsc

_BUFFER_SIZE = 100000
_D = 784
_MAXVAL = 256
_NUM_CHAINS = 1024
_N_NEW = 51
_N_OLD = _NUM_CHAINS - _N_NEW

_NC = 2
_NS = 16
_NW = _NC * _NS
_ROWS_PER_W = _NUM_CHAINS // _NW


_mesh = plsc.VectorSubcoreMesh(core_axis_name="c", subcore_axis_name="s")


@functools.partial(
    pl.kernel,
    mesh=_mesh,
    compiler_params=pltpu.CompilerParams(use_tc_tiling_on_sc=False),
    out_type=jax.ShapeDtypeStruct((_NUM_CHAINS, _D), jnp.int32),
    scratch_types=[
        pltpu.VMEM((_ROWS_PER_W,), jnp.int32),
        pltpu.VMEM((_ROWS_PER_W, _D), jnp.int32),
        pltpu.SemaphoreType.DMA,
    ],
)
def _assemble(buf_hbm, new_hbm, idx_hbm, out_hbm, idx_v, rows_v, sem):
    wid = lax.axis_index("s") * _NC + lax.axis_index("c")
    base = wid * _ROWS_PER_W
    pltpu.sync_copy(idx_hbm.at[pl.ds(base, _ROWS_PER_W)], idx_v)
    pltpu.async_copy(buf_hbm.at[idx_v], rows_v, sem).wait()

    @pl.when(wid == 0)
    def _():
        pltpu.sync_copy(new_hbm.at[pl.ds(0, _ROWS_PER_W)], rows_v)

    pltpu.sync_copy(rows_v, out_hbm.at[pl.ds(base, _ROWS_PER_W)])

    @pl.when(wid == 1)
    def _():
        pltpu.sync_copy(
            new_hbm.at[pl.ds(_ROWS_PER_W, _N_NEW - _ROWS_PER_W)],
            out_hbm.at[pl.ds(_ROWS_PER_W, _N_NEW - _ROWS_PER_W)],
        )


def kernel(buffer, key):
    key, subkey = jax.random.split(key, 2)
    new_samples = jax.random.randint(
        subkey, minval=0, maxval=_MAXVAL, shape=(_N_NEW, _D)
    )
    key, subkey = jax.random.split(key, 2)
    idx = jax.random.choice(subkey, _BUFFER_SIZE, shape=(_N_OLD,))
    idx_full = jnp.concatenate(
        [jnp.zeros((_N_NEW,), idx.dtype), idx.astype(jnp.int32)]
    )
    return _assemble(buffer, new_samples, idx_full)


# speedup vs baseline: 4.5560x; 4.5560x over previous
import functools

import jax
import jax.numpy as jnp
from jax import lax
from jax.experimental import pallas as pl
from jax.experimental.pallas import tpu as pltpu
from jax.experimental.pallas import tpu_sc as plsc

_BUFFER_SIZE = 100000
_D = 784
_MAXVAL = 256
_NUM_CHAINS = 1024
_N_NEW = 51
_N_OLD = _NUM_CHAINS - _N_NEW
_NEW_PAD = 56

_NC = 2
_NS = 16
_NW = _NC * _NS
_ROWS_PER_W = _NUM_CHAINS // _NW

_LANES = 16
_CHUNKS_PER_ROW = _D // _LANES
_RING = 4


_mesh = plsc.VectorSubcoreMesh(core_axis_name="c", subcore_axis_name="s")


@functools.partial(
    pl.kernel,
    mesh=_mesh,
    out_type=jax.ShapeDtypeStruct((_NUM_CHAINS, _D), jnp.int32),
    scratch_types=[
        pltpu.VMEM((_ROWS_PER_W,), jnp.int32),
        pltpu.VMEM((_ROWS_PER_W, _D), jnp.int32),
        pltpu.VMEM((8, _D), jnp.int32),
        pltpu.VMEM((8, _D), jnp.int32),
        pltpu.VMEM((8, _D), jnp.int32),
        pltpu.VMEM((8, _D), jnp.int32),
        pltpu.SemaphoreType.DMA,
    ],
)
def _assemble(
    buf_hbm, new_hbm, idx_hbm, out_hbm, idx_v, rows_v, b0, b1, b2, b3, sem
):
    blocks = [b0, b1, b2, b3]
    wid = lax.axis_index("s") * _NC + lax.axis_index("c")
    base = wid * _ROWS_PER_W
    pltpu.sync_copy(idx_hbm.at[pl.ds(base, _ROWS_PER_W)], idx_v)

    def gather_group(i, carry):
        v16 = idx_v[pl.ds(i * _LANES, _LANES)]
        for sub in range(_LANES // _RING):
            descs = []
            for b in range(_RING):
                v = v16[sub * _RING + b]
                blk = pl.multiple_of((v >> 3) << 3, 8)
                d = pltpu.async_copy(
                    buf_hbm.at[pl.ds(blk, 8)], blocks[b], sem
                )
                descs.append(d)
            for d in descs:
                d.wait()
            for b in range(_RING):
                r = v16[sub * _RING + b] & 7
                for c in range(_CHUNKS_PER_ROW):
                    rows_v[
                        i * _LANES + sub * _RING + b,
                        pl.ds(c * _LANES, _LANES),
                    ] = blocks[b][r, pl.ds(c * _LANES, _LANES)]
        return carry

    lax.fori_loop(0, _ROWS_PER_W // _LANES, gather_group, 0)

    @pl.when(wid == 0)
    def _():
        pltpu.sync_copy(new_hbm.at[pl.ds(0, _ROWS_PER_W)], rows_v)

    @pl.when(wid == 1)
    def _():
        pltpu.sync_copy(new_hbm.at[pl.ds(32, 16)], rows_v.at[pl.ds(0, 16)])
        pltpu.sync_copy(new_hbm.at[pl.ds(48, 8)], b0)
        for r in range(3):
            for c in range(_CHUNKS_PER_ROW):
                rows_v[16 + r, pl.ds(c * _LANES, _LANES)] = b0[
                    r, pl.ds(c * _LANES, _LANES)
                ]

    pltpu.sync_copy(rows_v, out_hbm.at[pl.ds(base, _ROWS_PER_W)])


def kernel(buffer, key):
    key, subkey = jax.random.split(key, 2)
    new_samples = jax.random.randint(
        subkey, minval=0, maxval=_MAXVAL, shape=(_N_NEW, _D)
    )
    key, subkey = jax.random.split(key, 2)
    idx = jax.random.choice(subkey, _BUFFER_SIZE, shape=(_N_OLD,))
    idx_full = jnp.concatenate(
        [jnp.zeros((_N_NEW,), idx.dtype), idx.astype(jnp.int32)]
    )
    new_padded = jnp.concatenate(
        [new_samples, jnp.zeros((_NEW_PAD - _N_NEW, _D), new_samples.dtype)]
    )
    return _assemble(buffer, new_padded, idx_full)
